# baseline (speedup 1.0000x reference)
import functools

import jax
import jax.numpy as jnp
from jax import lax
from jax.experimental import pallas as pl
from jax.experimental.pallas import tpu as pltpu
from jax.experimental.pallas import tpu_sc as plsc

VOCAB = 1000000
B = 4096
F = 100
ED = 16
SFN = F * ED
ATT = 30 * ED
MID = 100
AMS = MID + ATT
EPS = 1e-5
NIDX = B * F

_NW = 32
FP = 104
_GBR = 16
_GNC = (B // _NW) // _GBR


def _sc_gather(x2d, table, table_param, table2):
  mesh = plsc.VectorSubcoreMesh(core_axis_name="c", subcore_axis_name="s")

  @functools.partial(
      pl.kernel,
      mesh=mesh,
      compiler_params=pltpu.CompilerParams(use_tc_tiling_on_sc=False),
      out_type=(
          jax.ShapeDtypeStruct((B, FP, ED), jnp.float32),
          jax.ShapeDtypeStruct((B, FP, ED), jnp.float32),
          jax.ShapeDtypeStruct((B, F), jnp.float32),
      ),
      scratch_types=[
          pltpu.VMEM((_GBR, F), jnp.int32),
          pltpu.VMEM((_GBR, FP, ED), jnp.float32),
          pltpu.VMEM((_GBR, FP, ED), jnp.float32),
          pltpu.VMEM((_GBR, F), jnp.float32),
          pltpu.SemaphoreType.DMA,
          pltpu.SemaphoreType.DMA,
          pltpu.SemaphoreType.DMA,
      ],
  )
  def gather_k(x_hbm, t1_hbm, t2_hbm, t3_hbm, o1_hbm, o2_hbm, o3_hbm,
               idx_v, r1_v, r2_v, r3_v, s1, s2, s3):
    wid = lax.axis_index("s") * 2 + lax.axis_index("c")
    b00 = wid * (B // _NW)

    def body(c, carry):
      b0 = b00 + c * _GBR
      pltpu.sync_copy(x_hbm.at[pl.ds(b0, _GBR)], idx_v)
      cps = []
      for r in range(_GBR):
        cps.append(pltpu.async_copy(t1_hbm.at[idx_v.at[r]],
                                    r1_v.at[r, pl.ds(0, F)], s1))
        cps.append(pltpu.async_copy(t2_hbm.at[idx_v.at[r]],
                                    r2_v.at[r, pl.ds(0, F)], s2))
        cps.append(pltpu.async_copy(t3_hbm.at[idx_v.at[r]],
                                    r3_v.at[r], s3))
      for cp in cps:
        cp.wait()
      pltpu.sync_copy(r1_v, o1_hbm.at[pl.ds(b0, _GBR)])
      pltpu.sync_copy(r2_v, o2_hbm.at[pl.ds(b0, _GBR)])
      pltpu.sync_copy(r3_v, o3_hbm.at[pl.ds(b0, _GBR)])
      return carry

    lax.fori_loop(0, _GNC, body, 0)

  return gather_k(x2d, table, table_param, table2.reshape(VOCAB))


_TNC = (VOCAB + 127) // 128
_OROWS = VOCAB * ED // 128


_TG = 8
_TW = 128
_TNB = (_TNC - 1) * 128 // _TW


def _sc_transpose(tT1, tT2, tail1, tail2):
  mesh = plsc.VectorSubcoreMesh(core_axis_name="c", subcore_axis_name="s")

  @functools.partial(
      pl.kernel,
      mesh=mesh,
      compiler_params=pltpu.CompilerParams(needs_layout_passes=False),
      out_type=(
          jax.ShapeDtypeStruct((_OROWS, 128), jnp.float32),
          jax.ShapeDtypeStruct((_OROWS, 128), jnp.float32),
      ),
      scratch_types=[
          pltpu.VMEM((_TG, ED, _TW), jnp.float32),
          pltpu.VMEM((_TG, _TW * ED // 128, 128), jnp.float32),
          pltpu.SemaphoreType.DMA,
          pltpu.SemaphoreType.DMA,
      ],
  )
  def transpose_k(a_hbm, b_hbm, ta_hbm, tb_hbm, oa_hbm, ob_hbm,
                  tv, ov, si, so):
    wid = lax.axis_index("s") * 2 + lax.axis_index("c")
    c0 = wid * _TNB // _NW
    c1 = (wid + 1) * _TNB // _NW
    d_idx = lax.iota(jnp.int32, 16)

    row_base = d_idx // 8
    col_base = 16 * (d_idx % 8)
    row_vs = [row_base + 2 * t for t in range(_TW // 16)]
    col_vs = [col_base + d for d in range(ED)]

    def transpose_buf(b, valid):
      for t in range(valid // 16):
        for d in range(ED):
          vec = tv[b, d, 16 * t:16 * (t + 1)]
          plsc.store_scatter(ov.at[b], [row_vs[t], col_vs[d]], vec)

    orows = _TW * ED // 128

    def in_cp(src, c, b):
      return pltpu.make_async_copy(src.at[:, pl.ds(c * _TW, _TW)],
                                   tv.at[b], si)

    def out_cp(dst, c, b):
      return pltpu.make_async_copy(ov.at[b],
                                   dst.at[pl.ds(c * orows, orows), :], so)

    def do_table(src, dst):
      ngroups = (c1 - c0 + _TG - 1) // _TG

      def group(g, carry):
        cg = c0 + g * _TG
        for b in range(_TG):
          @pl.when(cg + b < c1)
          def _(b=b):
            in_cp(src, cg + b, b).start()
        for b in range(_TG):
          @pl.when(cg + b < c1)
          def _(b=b):
            in_cp(src, cg + b, b).wait()
            transpose_buf(b, _TW)
            out_cp(dst, cg + b, b).start()
        for b in range(_TG):
          @pl.when(cg + b < c1)
          def _(b=b):
            out_cp(dst, cg + b, b).wait()
        return carry

      lax.fori_loop(0, ngroups, group, 0)

    do_table(a_hbm, oa_hbm)
    do_table(b_hbm, ob_hbm)
    @pl.when(wid == _NW - 1)
    def _():
      for th, oh in ((ta_hbm, oa_hbm), (tb_hbm, ob_hbm)):
        pltpu.sync_copy(th, tv.at[0, :, pl.ds(0, 128)])
        transpose_buf(0, 64)
        pltpu.sync_copy(ov.at[0, pl.ds(0, 8)],
                        oh.at[pl.ds((_TNC - 1) * 16, 8), :])

  return transpose_k(tT1, tT2, tail1, tail2)


_BM = 512


def _gelu_exact(x):
  return 0.5 * x * (1.0 + lax.erf(x * 0.7071067811865476))


def _dense_body(emb_ref, embp_ref, emb1_ref,
                bns_ref, bnsh_ref, bnps_ref, bnpsh_ref,
                wfg_ref, bfg_ref, e_ref, wenc_ref, benc_ref,
                wmid_ref, bmid_ref, wfield_ref, bfield_ref, s_ref,
                w1_ref, b1_ref, wf2a_ref, wf2b_ref, bf2_ref,
                w2_ref, b2_ref, wf3a_ref, wf3b_ref, bf3_ref,
                w3_ref, b3_ref, out_ref):
  f32 = jnp.float32
  bf16 = jnp.bfloat16
  emb = emb_ref[...][:, :SFN]
  embp = embp_ref[...][:, :SFN]
  bno = emb * bns_ref[...] + bnsh_ref[...]
  bnop = embp * bnps_ref[...] + bnpsh_ref[...]
  bnop_h = bnop.astype(bf16)

  fg = jax.nn.sigmoid(jnp.dot(bnop_h, wfg_ref[...], preferred_element_type=f32)
                      + bfg_ref[...])
  fg_e = jnp.dot(fg.astype(bf16), e_ref[...], preferred_element_type=f32)
  bno_fg = fg_e * bnop
  bno_fg_h = bno_fg.astype(bf16)

  enc = jax.nn.sigmoid(jnp.dot(bno_fg_h, wenc_ref[...],
                               preferred_element_type=f32) + benc_ref[...])
  bn_att = bno_fg[:, :ATT]
  bno_d0 = jnp.concatenate([enc, bn_att], axis=1)
  mid = jax.nn.sigmoid(jnp.dot(bno_d0.astype(bf16), wmid_ref[...],
                               preferred_element_type=f32) + bmid_ref[...])
  bno_d = mid * bno_d0
  bno_d_h = bno_d.astype(bf16)
  param = jax.nn.sigmoid(jnp.dot(bno_d_h, wfield_ref[...],
                                 preferred_element_type=f32) + bfield_ref[...])
  new_emb = param * emb
  bno2 = param * bno

  s1 = jnp.dot(new_emb, s_ref[...], preferred_element_type=f32)
  s2 = jnp.dot(new_emb * new_emb, s_ref[...], preferred_element_type=f32)
  cross = 0.5 * jnp.sum(s1 * s1 - s2, axis=1, keepdims=True)

  d = _gelu_exact(jnp.dot(bno2.astype(bf16), w1_ref[...],
                          preferred_element_type=f32)
                  + b1_ref[...])
  p2 = jax.nn.sigmoid(jnp.dot(d.astype(bf16), wf2a_ref[...], preferred_element_type=f32)
                      + jnp.dot(bno_d_h, wf2b_ref[...], preferred_element_type=f32)
                      + bf2_ref[...])
  d = p2 * d
  d2 = _gelu_exact(jnp.dot(d.astype(bf16), w2_ref[...],
                           preferred_element_type=f32)
                   + b2_ref[...])
  p3 = jax.nn.sigmoid(jnp.dot(d2.astype(bf16), wf3a_ref[...], preferred_element_type=f32)
                      + jnp.dot(bno_d_h, wf3b_ref[...], preferred_element_type=f32)
                      + bf3_ref[...])
  d3 = p3 * d2
  out = jnp.dot(d3, w3_ref[...], preferred_element_type=f32) + b3_ref[...]
  e1 = jnp.sum(emb1_ref[...], axis=1, keepdims=True)
  out_ref[...] = jax.nn.sigmoid(out + cross + e1)


def _dense_forward(emb, embp, emb1, consts):
  grid = (B // _BM,)

  def blk(shape):
    nd = len(shape)
    return pl.BlockSpec(shape, lambda i: (i,) + (0,) * (nd - 1))

  def full(a):
    nd = a.ndim
    return pl.BlockSpec(a.shape, lambda i: (0,) * nd)

  in_specs = [blk((_BM, FP * ED)), blk((_BM, FP * ED)), blk((_BM, F))]
  in_specs += [full(c) for c in consts]
  return pl.pallas_call(
      _dense_body,
      grid=grid,
      in_specs=in_specs,
      out_specs=blk((_BM, 1)),
      out_shape=jax.ShapeDtypeStruct((B, 1), jnp.float32),
  )(emb, embp, emb1, *consts)


def kernel(x, table, table_param, table2, W1, b1, W2, b2, W3, b3,
           Wfg, bfg, Wenc, benc, Wmid, bmid, Wfield, bfield,
           Wf2, bf2, Wf3, bf3, bn_w, bn_b, bnp_w, bnp_b,
           bn_rm, bn_rv, bnp_rm, bnp_rv):
  tail_cols = VOCAB - (_TNC - 1) * 128
  pad = ((0, 0), (0, 128 - tail_cols))
  tail1 = jnp.pad(table.T[:, (_TNC - 1) * 128:], pad)
  tail2 = jnp.pad(table_param.T[:, (_TNC - 1) * 128:], pad)
  t1_lin, t2_lin = _sc_transpose(table.T, table_param.T, tail1, tail2)
  g1, g2, g3 = _sc_gather(x, t1_lin.reshape(VOCAB, ED),
                          t2_lin.reshape(VOCAB, ED), table2)
  emb = g1.reshape(B, FP * ED)
  embp = g2.reshape(B, FP * ED)
  emb1 = g3

  bns = (bn_w / jnp.sqrt(bn_rv + EPS)).reshape(1, SFN)
  bnsh = (bn_b - bn_rm * bns[0]).reshape(1, SFN)
  bnps = (bnp_w / jnp.sqrt(bnp_rv + EPS)).reshape(1, SFN)
  bnpsh = (bnp_b - bnp_rm * bnps[0]).reshape(1, SFN)

  E = jnp.repeat(jnp.eye(F, dtype=jnp.bfloat16), ED, axis=1)
  S = jnp.tile(jnp.eye(ED, dtype=jnp.float32), (F, 1))
  h = lambda w: w.astype(jnp.bfloat16)

  consts = (bns, bnsh, bnps, bnpsh,
            h(Wfg), bfg.reshape(1, F), E, h(Wenc), benc.reshape(1, MID),
            h(Wmid), bmid.reshape(1, AMS), h(Wfield), bfield.reshape(1, SFN), S,
            h(W1), b1.reshape(1, 1024), h(Wf2[:1024]), h(Wf2[1024:]), bf2.reshape(1, 1024),
            h(W2), b2.reshape(1, 512), h(Wf3[:512]), h(Wf3[512:]), bf3.reshape(1, 512),
            W3, b3.reshape(1, 1))
  return _dense_forward(emb, embp, emb1, consts)


# speedup vs baseline: 1.5133x; 1.5133x over previous
import functools

import jax
import jax.numpy as jnp
from jax import lax
from jax.experimental import pallas as pl
from jax.experimental.pallas import tpu as pltpu
from jax.experimental.pallas import tpu_sc as plsc

VOCAB = 1000000
B = 4096
F = 100
ED = 16
SFN = F * ED
ATT = 30 * ED
MID = 100
AMS = MID + ATT
EPS = 1e-5
NIDX = B * F

_NW = 32
_BPW = NIDX // _NW
_CH = 640
_NS = _CH // 128
_NCH = _BPW // _CH


def _sc_gather(idx2d, table, table_param, table2):
  mesh = plsc.VectorSubcoreMesh(core_axis_name="c", subcore_axis_name="s")

  @functools.partial(
      pl.kernel,
      mesh=mesh,
      compiler_params=pltpu.CompilerParams(use_tc_tiling_on_sc=False),
      out_type=(
          jax.ShapeDtypeStruct((NIDX, ED), jnp.float32),
          jax.ShapeDtypeStruct((NIDX, ED), jnp.float32),
          jax.ShapeDtypeStruct((NIDX,), jnp.float32),
      ),
      scratch_types=[
          pltpu.VMEM((_NS, 128), jnp.int32),
          pltpu.VMEM((_CH, ED), jnp.float32),
          pltpu.VMEM((_CH, ED), jnp.float32),
          pltpu.VMEM((_CH,), jnp.float32),
          pltpu.SemaphoreType.DMA,
          pltpu.SemaphoreType.DMA,
          pltpu.SemaphoreType.DMA,
      ],
  )
  def gather_k(idx_hbm, t1_hbm, t2_hbm, t3_hbm, o1_hbm, o2_hbm, o3_hbm,
               idx_v, r1_v, r2_v, r3_v, s1, s2, s3):
    wid = lax.axis_index("s") * 2 + lax.axis_index("c")
    base0 = wid * _BPW

    def body(c, carry):
      base = base0 + c * _CH
      pltpu.sync_copy(idx_hbm.at[pl.ds(base // 128, _NS)], idx_v)
      cps = []
      for j in range(_NS):
        dst = pl.ds(j * 128, 128)
        cps.append(pltpu.async_copy(t1_hbm.at[idx_v.at[j]], r1_v.at[dst], s1))
        cps.append(pltpu.async_copy(t2_hbm.at[idx_v.at[j]], r2_v.at[dst], s2))
        cps.append(pltpu.async_copy(t3_hbm.at[idx_v.at[j]], r3_v.at[dst], s3))
      for cp in cps:
        cp.wait()
      pltpu.sync_copy(r1_v, o1_hbm.at[pl.ds(base, _CH)])
      pltpu.sync_copy(r2_v, o2_hbm.at[pl.ds(base, _CH)])
      pltpu.sync_copy(r3_v, o3_hbm.at[pl.ds(base, _CH)])
      return carry

    lax.fori_loop(0, _NCH, body, 0)

  return gather_k(idx2d, table, table_param, table2.reshape(VOCAB))


_TNC = (VOCAB + 127) // 128
_OROWS = VOCAB * ED // 128


_TG = 8
_TW = 128
_TNB = (_TNC - 1) * 128 // _TW


def _sc_transpose(tT1, tT2, tail1, tail2):
  mesh = plsc.VectorSubcoreMesh(core_axis_name="c", subcore_axis_name="s")

  @functools.partial(
      pl.kernel,
      mesh=mesh,
      compiler_params=pltpu.CompilerParams(needs_layout_passes=False),
      out_type=(
          jax.ShapeDtypeStruct((_OROWS, 128), jnp.float32),
          jax.ShapeDtypeStruct((_OROWS, 128), jnp.float32),
      ),
      scratch_types=[
          pltpu.VMEM((_TG, ED, _TW), jnp.float32),
          pltpu.VMEM((_TG, _TW * ED // 128, 128), jnp.float32),
          pltpu.SemaphoreType.DMA,
          pltpu.SemaphoreType.DMA,
      ],
  )
  def transpose_k(a_hbm, b_hbm, ta_hbm, tb_hbm, oa_hbm, ob_hbm,
                  tv, ov, si, so):
    wid = lax.axis_index("s") * 2 + lax.axis_index("c")
    c0 = wid * _TNB // _NW
    c1 = (wid + 1) * _TNB // _NW
    d_idx = lax.iota(jnp.int32, 16)

    row_base = d_idx // 8
    col_base = 16 * (d_idx % 8)
    row_vs = [row_base + 2 * t for t in range(_TW // 16)]
    col_vs = [col_base + d for d in range(ED)]

    def transpose_buf(b, valid):
      for t in range(valid // 16):
        for d in range(ED):
          vec = tv[b, d, 16 * t:16 * (t + 1)]
          plsc.store_scatter(ov.at[b], [row_vs[t], col_vs[d]], vec)

    orows = _TW * ED // 128

    def in_cp(src, c, b):
      return pltpu.make_async_copy(src.at[:, pl.ds(c * _TW, _TW)],
                                   tv.at[b], si)

    def out_cp(dst, c, b):
      return pltpu.make_async_copy(ov.at[b],
                                   dst.at[pl.ds(c * orows, orows), :], so)

    def do_table(src, dst):
      ngroups = (c1 - c0 + _TG - 1) // _TG

      def group(g, carry):
        cg = c0 + g * _TG
        for b in range(_TG):
          @pl.when(cg + b < c1)
          def _(b=b):
            in_cp(src, cg + b, b).start()
        for b in range(_TG):
          @pl.when(cg + b < c1)
          def _(b=b):
            in_cp(src, cg + b, b).wait()
            transpose_buf(b, _TW)
            out_cp(dst, cg + b, b).start()
        for b in range(_TG):
          @pl.when(cg + b < c1)
          def _(b=b):
            out_cp(dst, cg + b, b).wait()
        return carry

      lax.fori_loop(0, ngroups, group, 0)

    do_table(a_hbm, oa_hbm)
    do_table(b_hbm, ob_hbm)
    @pl.when(wid == _NW - 1)
    def _():
      for th, oh in ((ta_hbm, oa_hbm), (tb_hbm, ob_hbm)):
        pltpu.sync_copy(th, tv.at[0, :, pl.ds(0, 128)])
        transpose_buf(0, 64)
        pltpu.sync_copy(ov.at[0, pl.ds(0, 8)],
                        oh.at[pl.ds((_TNC - 1) * 16, 8), :])

  return transpose_k(tT1, tT2, tail1, tail2)


_BM = 512


def _gelu_exact(x):
  return 0.5 * x * (1.0 + lax.erf(x * 0.7071067811865476))


def _dense_body(emb_ref, embp_ref, emb1_ref,
                bns_ref, bnsh_ref, bnps_ref, bnpsh_ref,
                wfg_ref, bfg_ref, e_ref, wenc_ref, benc_ref,
                wmid_ref, bmid_ref, wfield_ref, bfield_ref, s_ref,
                w1_ref, b1_ref, wf2a_ref, wf2b_ref, bf2_ref,
                w2_ref, b2_ref, wf3a_ref, wf3b_ref, bf3_ref,
                w3_ref, b3_ref, out_ref):
  f32 = jnp.float32
  bf16 = jnp.bfloat16
  emb = emb_ref[...]
  embp = embp_ref[...]
  bno = emb * bns_ref[...] + bnsh_ref[...]
  bnop = embp * bnps_ref[...] + bnpsh_ref[...]
  bnop_h = bnop.astype(bf16)

  fg = jax.nn.sigmoid(jnp.dot(bnop_h, wfg_ref[...], preferred_element_type=f32)
                      + bfg_ref[...])
  fg_e = jnp.dot(fg.astype(bf16), e_ref[...], preferred_element_type=f32)
  bno_fg = fg_e * bnop
  bno_fg_h = bno_fg.astype(bf16)

  enc = jax.nn.sigmoid(jnp.dot(bno_fg_h, wenc_ref[...],
                               preferred_element_type=f32) + benc_ref[...])
  bn_att = bno_fg[:, :ATT]
  bno_d0 = jnp.concatenate([enc, bn_att], axis=1)
  mid = jax.nn.sigmoid(jnp.dot(bno_d0.astype(bf16), wmid_ref[...],
                               preferred_element_type=f32) + bmid_ref[...])
  bno_d = mid * bno_d0
  bno_d_h = bno_d.astype(bf16)
  param = jax.nn.sigmoid(jnp.dot(bno_d_h, wfield_ref[...],
                                 preferred_element_type=f32) + bfield_ref[...])
  new_emb = param * emb
  bno2 = param * bno

  s1 = jnp.dot(new_emb, s_ref[...], preferred_element_type=f32)
  s2 = jnp.dot(new_emb * new_emb, s_ref[...], preferred_element_type=f32)
  cross = 0.5 * jnp.sum(s1 * s1 - s2, axis=1, keepdims=True)

  d = _gelu_exact(jnp.dot(bno2.astype(bf16), w1_ref[...],
                          preferred_element_type=f32)
                  + b1_ref[...])
  p2 = jax.nn.sigmoid(jnp.dot(d.astype(bf16), wf2a_ref[...], preferred_element_type=f32)
                      + jnp.dot(bno_d_h, wf2b_ref[...], preferred_element_type=f32)
                      + bf2_ref[...])
  d = p2 * d
  d2 = _gelu_exact(jnp.dot(d.astype(bf16), w2_ref[...],
                           preferred_element_type=f32)
                   + b2_ref[...])
  p3 = jax.nn.sigmoid(jnp.dot(d2.astype(bf16), wf3a_ref[...], preferred_element_type=f32)
                      + jnp.dot(bno_d_h, wf3b_ref[...], preferred_element_type=f32)
                      + bf3_ref[...])
  d3 = p3 * d2
  out = jnp.dot(d3, w3_ref[...], preferred_element_type=f32) + b3_ref[...]
  e1 = jnp.sum(emb1_ref[...], axis=1, keepdims=True)
  out_ref[...] = jax.nn.sigmoid(out + cross + e1)


def _dense_forward(emb, embp, emb1, consts):
  grid = (B // _BM,)

  def blk(shape):
    nd = len(shape)
    return pl.BlockSpec(shape, lambda i: (i,) + (0,) * (nd - 1))

  def full(a):
    nd = a.ndim
    return pl.BlockSpec(a.shape, lambda i: (0,) * nd)

  in_specs = [blk((_BM, SFN)), blk((_BM, SFN)), blk((_BM, F))]
  in_specs += [full(c) for c in consts]
  return pl.pallas_call(
      _dense_body,
      grid=grid,
      in_specs=in_specs,
      out_specs=blk((_BM, 1)),
      out_shape=jax.ShapeDtypeStruct((B, 1), jnp.float32),
  )(emb, embp, emb1, *consts)


def kernel(x, table, table_param, table2, W1, b1, W2, b2, W3, b3,
           Wfg, bfg, Wenc, benc, Wmid, bmid, Wfield, bfield,
           Wf2, bf2, Wf3, bf3, bn_w, bn_b, bnp_w, bnp_b,
           bn_rm, bn_rv, bnp_rm, bnp_rv):
  idx = x.reshape(NIDX // 128, 128)
  tail_cols = VOCAB - (_TNC - 1) * 128
  pad = ((0, 0), (0, 128 - tail_cols))
  tail1 = jnp.pad(table.T[:, (_TNC - 1) * 128:], pad)
  tail2 = jnp.pad(table_param.T[:, (_TNC - 1) * 128:], pad)
  t1_lin, t2_lin = _sc_transpose(table.T, table_param.T, tail1, tail2)
  g1, g2, g3 = _sc_gather(idx, t1_lin.reshape(VOCAB, ED),
                          t2_lin.reshape(VOCAB, ED), table2)
  emb = g1.reshape(B, SFN)
  embp = g2.reshape(B, SFN)
  emb1 = g3.reshape(B, F)

  bns = (bn_w / jnp.sqrt(bn_rv + EPS)).reshape(1, SFN)
  bnsh = (bn_b - bn_rm * bns[0]).reshape(1, SFN)
  bnps = (bnp_w / jnp.sqrt(bnp_rv + EPS)).reshape(1, SFN)
  bnpsh = (bnp_b - bnp_rm * bnps[0]).reshape(1, SFN)

  E = jnp.repeat(jnp.eye(F, dtype=jnp.bfloat16), ED, axis=1)
  S = jnp.tile(jnp.eye(ED, dtype=jnp.float32), (F, 1))
  h = lambda w: w.astype(jnp.bfloat16)

  consts = (bns, bnsh, bnps, bnpsh,
            h(Wfg), bfg.reshape(1, F), E, h(Wenc), benc.reshape(1, MID),
            h(Wmid), bmid.reshape(1, AMS), h(Wfield), bfield.reshape(1, SFN), S,
            h(W1), b1.reshape(1, 1024), h(Wf2[:1024]), h(Wf2[1024:]), bf2.reshape(1, 1024),
            h(W2), b2.reshape(1, 512), h(Wf3[:512]), h(Wf3[512:]), bf3.reshape(1, 512),
            W3, b3.reshape(1, 1))
  return _dense_forward(emb, embp, emb1, consts)


# speedup vs baseline: 1.8161x; 1.2001x over previous
import functools

import jax
import jax.numpy as jnp
from jax import lax
from jax.experimental import pallas as pl
from jax.experimental.pallas import tpu as pltpu
from jax.experimental.pallas import tpu_sc as plsc

VOCAB = 1000000
B = 4096
F = 100
ED = 16
SFN = F * ED
ATT = 30 * ED
MID = 100
AMS = MID + ATT
EPS = 1e-5
NIDX = B * F

_NW = 32
_BPW = NIDX // _NW
_CH = 640
_NS = _CH // 128
_NCH = _BPW // _CH


def _sc_gather(idx2d, table, table_param, table2):
  mesh = plsc.VectorSubcoreMesh(core_axis_name="c", subcore_axis_name="s")

  @functools.partial(
      pl.kernel,
      mesh=mesh,
      compiler_params=pltpu.CompilerParams(use_tc_tiling_on_sc=False),
      out_type=(
          jax.ShapeDtypeStruct((NIDX, ED), jnp.float32),
          jax.ShapeDtypeStruct((NIDX, ED), jnp.float32),
          jax.ShapeDtypeStruct((NIDX,), jnp.float32),
      ),
      scratch_types=[
          pltpu.VMEM((_NS, 128), jnp.int32),
          pltpu.VMEM((_CH, ED), jnp.float32),
          pltpu.VMEM((_CH, ED), jnp.float32),
          pltpu.VMEM((_CH,), jnp.float32),
          pltpu.SemaphoreType.DMA,
          pltpu.SemaphoreType.DMA,
          pltpu.SemaphoreType.DMA,
      ],
  )
  def gather_k(idx_hbm, t1_hbm, t2_hbm, t3_hbm, o1_hbm, o2_hbm, o3_hbm,
               idx_v, r1_v, r2_v, r3_v, s1, s2, s3):
    wid = lax.axis_index("s") * 2 + lax.axis_index("c")
    base0 = wid * _BPW

    def body(c, carry):
      base = base0 + c * _CH
      pltpu.sync_copy(idx_hbm.at[pl.ds(base // 128, _NS)], idx_v)
      cps = []
      for j in range(_NS):
        dst = pl.ds(j * 128, 128)
        cps.append(pltpu.async_copy(t1_hbm.at[idx_v.at[j]], r1_v.at[dst], s1))
        cps.append(pltpu.async_copy(t2_hbm.at[idx_v.at[j]], r2_v.at[dst], s2))
        cps.append(pltpu.async_copy(t3_hbm.at[idx_v.at[j]], r3_v.at[dst], s3))
      for cp in cps:
        cp.wait()
      pltpu.sync_copy(r1_v, o1_hbm.at[pl.ds(base, _CH)])
      pltpu.sync_copy(r2_v, o2_hbm.at[pl.ds(base, _CH)])
      pltpu.sync_copy(r3_v, o3_hbm.at[pl.ds(base, _CH)])
      return carry

    lax.fori_loop(0, _NCH, body, 0)

  return gather_k(idx2d, table, table_param, table2.reshape(VOCAB))


_TNC = (VOCAB + 127) // 128
_OROWS = VOCAB * ED // 128


_TG = 8
_TW = 128
_TNB = (_TNC - 1) * 128 // _TW


def _sc_transpose(tT1, tT2, tail1, tail2):
  mesh = plsc.VectorSubcoreMesh(core_axis_name="c", subcore_axis_name="s")

  @functools.partial(
      pl.kernel,
      mesh=mesh,
      compiler_params=pltpu.CompilerParams(needs_layout_passes=False),
      out_type=(
          jax.ShapeDtypeStruct((_OROWS, 128), jnp.float32),
          jax.ShapeDtypeStruct((_OROWS, 128), jnp.float32),
      ),
      scratch_types=[
          pltpu.VMEM((_TG, ED, _TW), jnp.float32),
          pltpu.VMEM((_TG, _TW * ED // 128, 128), jnp.float32),
          pltpu.SemaphoreType.DMA,
          pltpu.SemaphoreType.DMA,
      ],
  )
  def transpose_k(a_hbm, b_hbm, ta_hbm, tb_hbm, oa_hbm, ob_hbm,
                  tv, ov, si, so):
    wid = lax.axis_index("s") * 2 + lax.axis_index("c")
    c0 = wid * _TNB // _NW
    c1 = (wid + 1) * _TNB // _NW
    d_idx = lax.iota(jnp.int32, 16)

    row_base = d_idx // 8
    col_base = 16 * (d_idx % 8)
    row_vs = [row_base + 2 * t for t in range(_TW // 16)]
    col_vs = [col_base + d for d in range(ED)]

    def transpose_buf(b, valid):
      for t in range(valid // 16):
        for d in range(ED):
          vec = tv[b, d, 16 * t:16 * (t + 1)]
          plsc.store_scatter(ov.at[b], [row_vs[t], col_vs[d]], vec)

    orows = _TW * ED // 128

    def in_cp(src, c, b):
      return pltpu.make_async_copy(src.at[:, pl.ds(c * _TW, _TW)],
                                   tv.at[b], si)

    def out_cp(dst, c, b):
      return pltpu.make_async_copy(ov.at[b],
                                   dst.at[pl.ds(c * orows, orows), :], so)

    def do_table(src, dst):
      ngroups = (c1 - c0 + _TG - 1) // _TG

      for b in range(_TG):
        @pl.when(c0 + b < c1)
        def _(b=b):
          in_cp(src, c0 + b, b).start()

      def group(g, carry):
        cg = c0 + g * _TG
        for b in range(_TG):
          c = cg + b
          @pl.when(c < c1)
          def _(b=b, c=c):
            in_cp(src, c, b).wait()
            @pl.when(g > 0)
            def _():
              out_cp(dst, c - _TG, b).wait()
            transpose_buf(b, _TW)
            out_cp(dst, c, b).start()
            @pl.when(c + _TG < c1)
            def _():
              in_cp(src, c + _TG, b).start()
        return carry

      lax.fori_loop(0, ngroups, group, 0)
      last_cg = c0 + (ngroups - 1) * _TG
      for b in range(_TG):
        @pl.when(last_cg + b < c1)
        def _(b=b):
          out_cp(dst, last_cg + b, b).wait()

    do_table(a_hbm, oa_hbm)
    do_table(b_hbm, ob_hbm)
    @pl.when(wid == _NW - 1)
    def _():
      for th, oh in ((ta_hbm, oa_hbm), (tb_hbm, ob_hbm)):
        pltpu.sync_copy(th, tv.at[0, :, pl.ds(0, 128)])
        transpose_buf(0, 64)
        pltpu.sync_copy(ov.at[0, pl.ds(0, 8)],
                        oh.at[pl.ds((_TNC - 1) * 16, 8), :])

  return transpose_k(tT1, tT2, tail1, tail2)


_BM = 512


def _gelu_exact(x):
  return 0.5 * x * (1.0 + lax.erf(x * 0.7071067811865476))


def _dense_body(emb_ref, embp_ref, emb1_ref,
                bns_ref, bnsh_ref, bnps_ref, bnpsh_ref,
                wfg_ref, bfg_ref, e_ref, wenc_ref, benc_ref,
                wmid_ref, bmid_ref, wfield_ref, bfield_ref, s_ref,
                w1_ref, b1_ref, wf2a_ref, wf2b_ref, bf2_ref,
                w2_ref, b2_ref, wf3a_ref, wf3b_ref, bf3_ref,
                w3_ref, b3_ref, out_ref):
  f32 = jnp.float32
  bf16 = jnp.bfloat16
  emb = emb_ref[...]
  embp = embp_ref[...]
  bno = emb * bns_ref[...] + bnsh_ref[...]
  bnop = embp * bnps_ref[...] + bnpsh_ref[...]
  bnop_h = bnop.astype(bf16)

  fg = jax.nn.sigmoid(jnp.dot(bnop_h, wfg_ref[...], preferred_element_type=f32)
                      + bfg_ref[...])
  fg_e = jnp.dot(fg.astype(bf16), e_ref[...], preferred_element_type=f32)
  bno_fg = fg_e * bnop
  bno_fg_h = bno_fg.astype(bf16)

  enc = jax.nn.sigmoid(jnp.dot(bno_fg_h, wenc_ref[...],
                               preferred_element_type=f32) + benc_ref[...])
  bn_att = bno_fg[:, :ATT]
  bno_d0 = jnp.concatenate([enc, bn_att], axis=1)
  mid = jax.nn.sigmoid(jnp.dot(bno_d0.astype(bf16), wmid_ref[...],
                               preferred_element_type=f32) + bmid_ref[...])
  bno_d = mid * bno_d0
  bno_d_h = bno_d.astype(bf16)
  param = jax.nn.sigmoid(jnp.dot(bno_d_h, wfield_ref[...],
                                 preferred_element_type=f32) + bfield_ref[...])
  new_emb = param * emb
  bno2 = param * bno

  s1 = jnp.dot(new_emb, s_ref[...], preferred_element_type=f32)
  s2 = jnp.dot(new_emb * new_emb, s_ref[...], preferred_element_type=f32)
  cross = 0.5 * jnp.sum(s1 * s1 - s2, axis=1, keepdims=True)

  d = _gelu_exact(jnp.dot(bno2.astype(bf16), w1_ref[...],
                          preferred_element_type=f32)
                  + b1_ref[...])
  p2 = jax.nn.sigmoid(jnp.dot(d.astype(bf16), wf2a_ref[...], preferred_element_type=f32)
                      + jnp.dot(bno_d_h, wf2b_ref[...], preferred_element_type=f32)
                      + bf2_ref[...])
  d = p2 * d
  d2 = _gelu_exact(jnp.dot(d.astype(bf16), w2_ref[...],
                           preferred_element_type=f32)
                   + b2_ref[...])
  p3 = jax.nn.sigmoid(jnp.dot(d2.astype(bf16), wf3a_ref[...], preferred_element_type=f32)
                      + jnp.dot(bno_d_h, wf3b_ref[...], preferred_element_type=f32)
                      + bf3_ref[...])
  d3 = p3 * d2
  out = jnp.dot(d3, w3_ref[...], preferred_element_type=f32) + b3_ref[...]
  e1 = jnp.sum(emb1_ref[...], axis=1, keepdims=True)
  out_ref[...] = jax.nn.sigmoid(out + cross + e1)


def _dense_forward(emb, embp, emb1, consts):
  grid = (B // _BM,)

  def blk(shape):
    nd = len(shape)
    return pl.BlockSpec(shape, lambda i: (i,) + (0,) * (nd - 1))

  def full(a):
    nd = a.ndim
    return pl.BlockSpec(a.shape, lambda i: (0,) * nd)

  in_specs = [blk((_BM, SFN)), blk((_BM, SFN)), blk((_BM, F))]
  in_specs += [full(c) for c in consts]
  return pl.pallas_call(
      _dense_body,
      grid=grid,
      in_specs=in_specs,
      out_specs=blk((_BM, 1)),
      out_shape=jax.ShapeDtypeStruct((B, 1), jnp.float32),
  )(emb, embp, emb1, *consts)


def kernel(x, table, table_param, table2, W1, b1, W2, b2, W3, b3,
           Wfg, bfg, Wenc, benc, Wmid, bmid, Wfield, bfield,
           Wf2, bf2, Wf3, bf3, bn_w, bn_b, bnp_w, bnp_b,
           bn_rm, bn_rv, bnp_rm, bnp_rv):
  idx = x.reshape(NIDX // 128, 128)
  tail_cols = VOCAB - (_TNC - 1) * 128
  pad = ((0, 0), (0, 128 - tail_cols))
  tail1 = jnp.pad(table.T[:, (_TNC - 1) * 128:], pad)
  tail2 = jnp.pad(table_param.T[:, (_TNC - 1) * 128:], pad)
  t1_lin, t2_lin = _sc_transpose(table.T, table_param.T, tail1, tail2)
  g1, g2, g3 = _sc_gather(idx, t1_lin.reshape(VOCAB, ED),
                          t2_lin.reshape(VOCAB, ED), table2)
  emb = g1.reshape(B, SFN)
  embp = g2.reshape(B, SFN)
  emb1 = g3.reshape(B, F)

  bns = (bn_w / jnp.sqrt(bn_rv + EPS)).reshape(1, SFN)
  bnsh = (bn_b - bn_rm * bns[0]).reshape(1, SFN)
  bnps = (bnp_w / jnp.sqrt(bnp_rv + EPS)).reshape(1, SFN)
  bnpsh = (bnp_b - bnp_rm * bnps[0]).reshape(1, SFN)

  E = jnp.repeat(jnp.eye(F, dtype=jnp.bfloat16), ED, axis=1)
  S = jnp.tile(jnp.eye(ED, dtype=jnp.float32), (F, 1))
  h = lambda w: w.astype(jnp.bfloat16)

  consts = (bns, bnsh, bnps, bnpsh,
            h(Wfg), bfg.reshape(1, F), E, h(Wenc), benc.reshape(1, MID),
            h(Wmid), bmid.reshape(1, AMS), h(Wfield), bfield.reshape(1, SFN), S,
            h(W1), b1.reshape(1, 1024), h(Wf2[:1024]), h(Wf2[1024:]), bf2.reshape(1, 1024),
            h(W2), b2.reshape(1, 512), h(Wf3[:512]), h(Wf3[512:]), bf3.reshape(1, 512),
            W3, b3.reshape(1, 1))
  return _dense_forward(emb, embp, emb1, consts)
